# baseline (speedup 1.0000x reference)
import functools
import math

import jax
import jax.numpy as jnp
from jax import lax
from jax.experimental import pallas as pl
from jax.experimental.pallas import tpu as pltpu
from jax.experimental.pallas import tpu_sc as plsc

EMBD = 128
D = 1024
NH = 16
DH = 64
NE = 64
DFF = 2048
NLAYERS = 2
LNEPS = 1e-12
T = 2048
CAP = 64
NSLOT = NE * CAP
QB = 256
TB = 256
HB = 128
NSLOTP = NSLOT + 128

_BF = jnp.bfloat16
_F32 = jnp.float32


def _ln(x, g, b):
    m = jnp.mean(x, axis=-1, keepdims=True)
    v = jnp.mean((x - m) ** 2, axis=-1, keepdims=True)
    return (x - m) / jnp.sqrt(v + LNEPS) * g + b


def _embed_kernel(x_ref, w_ref, b_ref, o_ref, obf_ref):
    h = jnp.dot(x_ref[...], w_ref[...], preferred_element_type=_F32) + b_ref[...]
    o_ref[...] = h
    obf_ref[...] = h.astype(_BF)


def _embed(x, w, b):
    return pl.pallas_call(
        _embed_kernel,
        out_shape=(
            jax.ShapeDtypeStruct((T, D), _F32),
            jax.ShapeDtypeStruct((T, D), _BF),
        ),
    )(x, w, b)


def _attn_kernel(hbf_ref, wq_ref, bq_ref, wk_ref, bk_ref, wv_ref, bv_ref,
                 ctx_ref, q_sc, k_sc, v_sc):
    hbf = hbf_ref[...]
    q = jnp.dot(hbf, wq_ref[...].astype(_BF), preferred_element_type=_F32) + bq_ref[...]
    k = jnp.dot(hbf, wk_ref[...].astype(_BF), preferred_element_type=_F32) + bk_ref[...]
    v = jnp.dot(hbf, wv_ref[...].astype(_BF), preferred_element_type=_F32) + bv_ref[...]
    q_sc[...] = (q * (1.0 / math.sqrt(DH))).astype(_BF)
    k_sc[...] = k.astype(_BF)
    v_sc[...] = v.astype(_BF)
    ones_col = (jax.lax.broadcasted_iota(jnp.int32, (T, DH), 1) == 0
                ).astype(_BF)
    for sub in range(2):
        lo = sub * DH
        kh = k_sc[:, lo:lo + DH]
        vaug = jnp.concatenate([v_sc[:, lo:lo + DH], ones_col], axis=1)

        def body(i, _):
            qh = q_sc[pl.ds(i * QB, QB), lo:lo + DH]
            s = jax.lax.dot_general(qh, kh, (((1,), (1,)), ((), ())),
                                    preferred_element_type=_F32)
            e = jnp.exp(jnp.minimum(s, 30.0).astype(_BF))
            cb = jnp.dot(e, vaug, preferred_element_type=_F32)
            ctx_ref[pl.ds(i * QB, QB), lo:lo + DH] = (
                cb[:, :DH] / cb[:, DH:DH + 1]).astype(_BF)
            return 0

        jax.lax.fori_loop(0, T // QB, body, 0)


def _attn(hbf, wq, bq, wk, bk, wv, bv):
    return pl.pallas_call(
        _attn_kernel,
        grid=(D // HB,),
        in_specs=[
            pl.BlockSpec((T, D), lambda i: (0, 0)),
            pl.BlockSpec((D, HB), lambda i: (0, i)),
            pl.BlockSpec((1, HB), lambda i: (0, i)),
            pl.BlockSpec((D, HB), lambda i: (0, i)),
            pl.BlockSpec((1, HB), lambda i: (0, i)),
            pl.BlockSpec((D, HB), lambda i: (0, i)),
            pl.BlockSpec((1, HB), lambda i: (0, i)),
        ],
        out_specs=pl.BlockSpec((T, HB), lambda i: (0, i)),
        out_shape=jax.ShapeDtypeStruct((T, D), _BF),
        scratch_shapes=[
            pltpu.VMEM((T, HB), _BF),
            pltpu.VMEM((T, HB), _BF),
            pltpu.VMEM((T, HB), _BF),
        ],
        compiler_params=pltpu.CompilerParams(dimension_semantics=("arbitrary",)),
    )(hbf, wq, bq, wk, bk, wv, bv)


def _router_kernel(ctx_ref, wo_ref, bo_ref, h_ref, g_ref, b_ref, wr_ref,
                   a_ref, abf_ref, code_ref, code2_ref, scale_ref, loss_ref):
    att = jnp.dot(ctx_ref[...], wo_ref[...].astype(_BF),
                  preferred_element_type=_F32) + bo_ref[...] + h_ref[...]
    a = _ln(att, g_ref[...], b_ref[...])
    a_ref[...] = a
    abf_ref[...] = a.astype(_BF)
    logits = jnp.dot(a, wr_ref[...], preferred_element_type=_F32)
    m = jnp.max(logits, axis=-1, keepdims=True)
    ex = jnp.exp(logits - m)
    se = jnp.sum(ex, axis=-1, keepdims=True)
    ii = jax.lax.broadcasted_iota(jnp.int32, (T, NE), 1)
    eidx = jnp.min(jnp.where(logits == m, ii, NE), axis=-1, keepdims=True)
    gate = 1.0 / se
    onehot = (ii == eidx).astype(jnp.int32)
    cs = onehot
    sh = 1
    while sh < T:
        cs = cs + jnp.concatenate(
            [jnp.zeros((sh, NE), jnp.int32), cs[: T - sh, :]], axis=0)
        sh *= 2
    pos = jnp.sum(cs * onehot, axis=-1, keepdims=True) - 1
    keep = pos < CAP
    slot = jnp.where(keep, pos, CAP)
    tt = jax.lax.broadcasted_iota(jnp.int32, (T, 1), 0)
    code_ref[...] = jnp.where(keep, eidx * CAP + slot, NSLOT + (tt % 128))
    code2_ref[...] = eidx * CAP + jnp.clip(slot, 0, CAP - 1)
    scale_ref[...] = gate * keep.astype(_F32)
    probs = ex / se
    f = jnp.mean(onehot.astype(_F32), axis=0, keepdims=True)
    pm = jnp.mean(probs, axis=0, keepdims=True)
    loss_ref[...] = jnp.sum(f * pm, axis=-1, keepdims=True) * NE


def _router(ctx, wo, bo, h, g, b, wr):
    return pl.pallas_call(
        _router_kernel,
        out_shape=(
            jax.ShapeDtypeStruct((T, D), _F32),
            jax.ShapeDtypeStruct((T, D), _BF),
            jax.ShapeDtypeStruct((T, 1), jnp.int32),
            jax.ShapeDtypeStruct((T, 1), jnp.int32),
            jax.ShapeDtypeStruct((T, 1), _F32),
            jax.ShapeDtypeStruct((1, 1), _F32),
        ),
    )(ctx, wo, bo, h, g, b, wr)


def _wid():
    return lax.axis_index("s") * 2 + lax.axis_index("c")


_DCH = 64
_sc_cache = {}


def _sc_kernels():
    if _sc_cache:
        return _sc_cache["disp"], _sc_cache["comb"]
    mesh = plsc.VectorSubcoreMesh(core_axis_name="c", subcore_axis_name="s")

    @functools.partial(
        pl.kernel,
        out_type=jax.ShapeDtypeStruct((NSLOTP, D // 2), jnp.int32),
        mesh=mesh,
        scratch_types=[
            pltpu.VMEM((T // 32,), jnp.int32),
            pltpu.VMEM((T // 32, D // 2), jnp.int32),
            pltpu.SemaphoreType.DMA,
        ],
    )
    def disp(a_hbm, code_hbm, buf_hbm, idx_v, rows_v, sem):
        base = _wid() * (T // 32)
        pltpu.sync_copy(code_hbm.at[pl.ds(base, T // 32)], idx_v)
        pltpu.sync_copy(a_hbm.at[pl.ds(base, T // 32)], rows_v)
        pltpu.async_copy(rows_v, buf_hbm.at[idx_v], sem).wait()

    @functools.partial(
        pl.kernel,
        out_type=jax.ShapeDtypeStruct((T, D // 2), jnp.int32),
        mesh=mesh,
        scratch_types=[
            pltpu.VMEM((T // 32,), jnp.int32),
            pltpu.VMEM((T // 32, D // 2), jnp.int32),
            pltpu.SemaphoreType.DMA,
        ],
    )
    def comb(y_hbm, code2_hbm, out_hbm, idx_v, rows_v, sem):
        base = _wid() * (T // 32)
        pltpu.sync_copy(code2_hbm.at[pl.ds(base, T // 32)], idx_v)
        pltpu.async_copy(y_hbm.at[idx_v], rows_v, sem).wait()
        pltpu.sync_copy(rows_v, out_hbm.at[pl.ds(base, T // 32)])

    _sc_cache.update(disp=disp, comb=comb)
    return disp, comb


def _ffn_kernel(buf_ref, w1_ref, b1_ref, w2_ref, b2_ref, o_ref):
    xb = buf_ref[...]
    hh = jnp.dot(xb, w1_ref[0].astype(_BF), preferred_element_type=_F32) + b1_ref[0]
    hh = jax.nn.gelu(hh)
    o_ref[...] = (jnp.dot(hh.astype(_BF), w2_ref[0].astype(_BF),
                          preferred_element_type=_F32) + b2_ref[0]).astype(_BF)


def _ffn(buf, w1, b1, w2, b2):
    return pl.pallas_call(
        _ffn_kernel,
        grid=(NE,),
        in_specs=[
            pl.BlockSpec((CAP, D), lambda e: (e, 0)),
            pl.BlockSpec((1, D, DFF), lambda e: (e, 0, 0)),
            pl.BlockSpec((1, 1, DFF), lambda e: (e, 0, 0)),
            pl.BlockSpec((1, DFF, D), lambda e: (e, 0, 0)),
            pl.BlockSpec((1, 1, D), lambda e: (e, 0, 0)),
        ],
        out_specs=pl.BlockSpec((CAP, D), lambda e: (e, 0)),
        out_shape=jax.ShapeDtypeStruct((NSLOT, D), _BF),
        compiler_params=pltpu.CompilerParams(dimension_semantics=("parallel",)),
    )(buf, w1, b1, w2, b2)


def _lnout_kernel(gath_ref, scale_ref, a_ref, g_ref, b_ref, o_ref, obf_ref):
    ffn = gath_ref[...].astype(_F32) * scale_ref[...]
    out = _ln(ffn + a_ref[...], g_ref[...], b_ref[...])
    o_ref[...] = out
    obf_ref[...] = out.astype(_BF)


def _lnout(gath, scale, apad, g, b):
    return pl.pallas_call(
        _lnout_kernel,
        grid=(T // TB,),
        in_specs=[
            pl.BlockSpec((TB, D), lambda i: (i, 0)),
            pl.BlockSpec((TB, 1), lambda i: (i, 0)),
            pl.BlockSpec((TB, D), lambda i: (i, 0)),
            pl.BlockSpec((1, D), lambda i: (0, 0)),
            pl.BlockSpec((1, D), lambda i: (0, 0)),
        ],
        out_specs=(
            pl.BlockSpec((TB, D), lambda i: (i, 0)),
            pl.BlockSpec((TB, D), lambda i: (i, 0)),
        ),
        out_shape=(
            jax.ShapeDtypeStruct((T, D), _F32),
            jax.ShapeDtypeStruct((T, D), _BF),
        ),
        compiler_params=pltpu.CompilerParams(dimension_semantics=("arbitrary",)),
    )(gath, scale, apad, g, b)


def _b2i(x):
    return jax.lax.bitcast_convert_type(
        x.reshape(x.shape[0], D // 2, 2), jnp.int32)


def _i2b(x):
    return jax.lax.bitcast_convert_type(x, _BF).reshape(x.shape[0], -1)


def kernel(hidden_states, W_emb, b_emb, Wq, bq, Wk, bk, Wv, bv, Wo, bo,
           ln_attn_g, ln_attn_b, Wr, W1, b1, W2, b2, ln_out_g, ln_out_b):
    x = hidden_states.reshape(T, EMBD)
    r2 = lambda v: v.reshape(1, -1)
    b1h = b1.reshape(NE, 1, DFF)
    b2h = b2.reshape(NE, 1, D)
    h, hbf = _embed(x, W_emb, r2(b_emb))
    losses = []
    for _ in range(NLAYERS):
        ctx = _attn(hbf, Wq, r2(bq), Wk, r2(bk), Wv, r2(bv))
        a, abf, code, code2, scale, loss = _router(
            ctx, Wo, r2(bo), h, r2(ln_attn_g), r2(ln_attn_b), Wr)
        sc_dispatch, sc_combine = _sc_kernels()
        buf = sc_dispatch(_b2i(abf), code.reshape(T))
        y = _ffn(_i2b(buf), W1, b1h, W2, b2h)
        gath = sc_combine(_b2i(y), code2.reshape(T))
        h, hbf = _lnout(_i2b(gath), scale, a, r2(ln_out_g), r2(ln_out_b))
        losses.append(loss[0, 0])
    return h.reshape(1, T, D), jnp.stack(losses)


# speedup vs baseline: 1.4952x; 1.4952x over previous
import functools
import math

import jax
import jax.numpy as jnp
from jax import lax
from jax.experimental import pallas as pl
from jax.experimental.pallas import tpu as pltpu
from jax.experimental.pallas import tpu_sc as plsc

EMBD = 128
D = 1024
NH = 16
DH = 64
NE = 64
DFF = 2048
NLAYERS = 2
LNEPS = 1e-12
T = 2048
CAP = 64
NSLOT = NE * CAP
QB = 256
TB = 256
HB = 128
NSLOTP = NSLOT + 128

_BF = jnp.bfloat16
_F32 = jnp.float32


def _ln(x, g, b):
    m = jnp.mean(x, axis=-1, keepdims=True)
    v = jnp.mean((x - m) ** 2, axis=-1, keepdims=True)
    return (x - m) / jnp.sqrt(v + LNEPS) * g + b


def _embed_kernel(x_ref, w_ref, b_ref, o_ref, obf_ref):
    h = jnp.dot(x_ref[...], w_ref[...], preferred_element_type=_F32) + b_ref[...]
    o_ref[...] = h
    obf_ref[...] = h.astype(_BF)


def _embed(x, w, b):
    return pl.pallas_call(
        _embed_kernel,
        out_shape=(
            jax.ShapeDtypeStruct((T, D), _F32),
            jax.ShapeDtypeStruct((T, D), _BF),
        ),
    )(x, w, b)


def _attn_kernel(hbf_ref, wq_ref, bq_ref, wk_ref, bk_ref, wv_ref, bv_ref,
                 ctx_ref, q_sc, k_sc, v_sc):
    hbf = hbf_ref[...]
    q = jnp.dot(hbf, wq_ref[...].astype(_BF), preferred_element_type=_F32) + bq_ref[...]
    k = jnp.dot(hbf, wk_ref[...].astype(_BF), preferred_element_type=_F32) + bk_ref[...]
    v = jnp.dot(hbf, wv_ref[...].astype(_BF), preferred_element_type=_F32) + bv_ref[...]
    q_sc[...] = (q * (1.0 / math.sqrt(DH))).astype(_BF)
    k_sc[...] = k.astype(_BF)
    v_sc[...] = v.astype(_BF)
    ones_col = (jax.lax.broadcasted_iota(jnp.int32, (T, DH), 1) == 0
                ).astype(_BF)
    for sub in range(2):
        lo = sub * DH
        kh = k_sc[:, lo:lo + DH]
        vaug = jnp.concatenate([v_sc[:, lo:lo + DH], ones_col], axis=1)

        def body(i, _):
            qh = q_sc[pl.ds(i * QB, QB), lo:lo + DH]
            s = jax.lax.dot_general(qh, kh, (((1,), (1,)), ((), ())),
                                    preferred_element_type=_F32)
            e = jnp.exp(jnp.minimum(s, 30.0).astype(_BF))
            cb = jnp.dot(e, vaug, preferred_element_type=_F32)
            ctx_ref[pl.ds(i * QB, QB), lo:lo + DH] = (
                cb[:, :DH] / cb[:, DH:DH + 1]).astype(_BF)
            return 0

        jax.lax.fori_loop(0, T // QB, body, 0)


def _attn(hbf, wq, bq, wk, bk, wv, bv):
    return pl.pallas_call(
        _attn_kernel,
        grid=(D // HB,),
        in_specs=[
            pl.BlockSpec((T, D), lambda i: (0, 0)),
            pl.BlockSpec((D, HB), lambda i: (0, i)),
            pl.BlockSpec((1, HB), lambda i: (0, i)),
            pl.BlockSpec((D, HB), lambda i: (0, i)),
            pl.BlockSpec((1, HB), lambda i: (0, i)),
            pl.BlockSpec((D, HB), lambda i: (0, i)),
            pl.BlockSpec((1, HB), lambda i: (0, i)),
        ],
        out_specs=pl.BlockSpec((T, HB), lambda i: (0, i)),
        out_shape=jax.ShapeDtypeStruct((T, D), _BF),
        scratch_shapes=[
            pltpu.VMEM((T, HB), _BF),
            pltpu.VMEM((T, HB), _BF),
            pltpu.VMEM((T, HB), _BF),
        ],
        compiler_params=pltpu.CompilerParams(dimension_semantics=("arbitrary",)),
    )(hbf, wq, bq, wk, bk, wv, bv)


def _router_kernel(ctx_ref, wo_ref, bo_ref, h_ref, g_ref, b_ref, wr_ref,
                   a_ref, code_ref, code2_ref, scale_ref, loss_ref):
    att = jnp.dot(ctx_ref[...], wo_ref[...].astype(_BF),
                  preferred_element_type=_F32) + bo_ref[...] + h_ref[...]
    a = _ln(att, g_ref[...], b_ref[...])
    a_ref[...] = a
    logits = jnp.dot(a, wr_ref[...], preferred_element_type=_F32)
    m = jnp.max(logits, axis=-1, keepdims=True)
    ex = jnp.exp(logits - m)
    se = jnp.sum(ex, axis=-1, keepdims=True)
    ii = jax.lax.broadcasted_iota(jnp.int32, (T, NE), 1)
    eidx = jnp.min(jnp.where(logits == m, ii, NE), axis=-1, keepdims=True)
    gate = 1.0 / se
    onehot = (ii == eidx).astype(jnp.int32)
    cs = onehot
    sh = 1
    while sh < T:
        cs = cs + jnp.concatenate(
            [jnp.zeros((sh, NE), jnp.int32), cs[: T - sh, :]], axis=0)
        sh *= 2
    pos = jnp.sum(cs * onehot, axis=-1, keepdims=True) - 1
    keep = pos < CAP
    slot = jnp.where(keep, pos, CAP)
    tt = jax.lax.broadcasted_iota(jnp.int32, (T, 1), 0)
    code_ref[...] = jnp.where(keep, eidx * CAP + slot, NSLOT + (tt % 128))
    code2_ref[...] = eidx * CAP + jnp.clip(slot, 0, CAP - 1)
    scale_ref[...] = gate * keep.astype(_F32)
    probs = ex / se
    f = jnp.mean(onehot.astype(_F32), axis=0, keepdims=True)
    pm = jnp.mean(probs, axis=0, keepdims=True)
    loss_ref[...] = jnp.sum(f * pm, axis=-1, keepdims=True) * NE


def _router(ctx, wo, bo, h, g, b, wr):
    return pl.pallas_call(
        _router_kernel,
        out_shape=(
            jax.ShapeDtypeStruct((T, D), _F32),
            jax.ShapeDtypeStruct((T, 1), jnp.int32),
            jax.ShapeDtypeStruct((T, 1), jnp.int32),
            jax.ShapeDtypeStruct((T, 1), _F32),
            jax.ShapeDtypeStruct((1, 1), _F32),
        ),
    )(ctx, wo, bo, h, g, b, wr)


def _wid():
    return lax.axis_index("s") * 2 + lax.axis_index("c")


_DCH = 64
_sc_cache = {}


def _sc_kernels():
    if _sc_cache:
        return _sc_cache["disp"], _sc_cache["comb"]
    mesh = plsc.VectorSubcoreMesh(core_axis_name="c", subcore_axis_name="s")

    @functools.partial(
        pl.kernel,
        out_type=jax.ShapeDtypeStruct((NSLOTP, D), _F32),
        mesh=mesh,
        scratch_types=[
            pltpu.VMEM((T // 32,), jnp.int32),
            pltpu.VMEM((T // 32, D), _F32),
            pltpu.SemaphoreType.DMA,
        ],
    )
    def disp(a_hbm, code_hbm, buf_hbm, idx_v, rows_v, sem):
        base = _wid() * (T // 32)
        pltpu.sync_copy(code_hbm.at[pl.ds(base, T // 32)], idx_v)
        pltpu.sync_copy(a_hbm.at[pl.ds(base, T // 32)], rows_v)
        pltpu.async_copy(rows_v, buf_hbm.at[idx_v], sem).wait()

    @functools.partial(
        pl.kernel,
        out_type=jax.ShapeDtypeStruct((T, D), _F32),
        mesh=mesh,
        scratch_types=[
            pltpu.VMEM((T // 32,), jnp.int32),
            pltpu.VMEM((T // 32, D), _F32),
            pltpu.SemaphoreType.DMA,
        ],
    )
    def comb(y_hbm, code2_hbm, out_hbm, idx_v, rows_v, sem):
        base = _wid() * (T // 32)
        pltpu.sync_copy(code2_hbm.at[pl.ds(base, T // 32)], idx_v)
        pltpu.async_copy(y_hbm.at[idx_v], rows_v, sem).wait()
        pltpu.sync_copy(rows_v, out_hbm.at[pl.ds(base, T // 32)])

    _sc_cache.update(disp=disp, comb=comb)
    return disp, comb


def _ffn_kernel(buf_ref, w1_ref, b1_ref, w2_ref, b2_ref, o_ref):
    xb = buf_ref[...].astype(_BF)
    hh = jnp.dot(xb, w1_ref[0].astype(_BF), preferred_element_type=_F32) + b1_ref[0]
    hh = jax.nn.gelu(hh)
    o_ref[...] = jnp.dot(hh.astype(_BF), w2_ref[0].astype(_BF),
                         preferred_element_type=_F32) + b2_ref[0]


def _ffn(buf, w1, b1, w2, b2):
    return pl.pallas_call(
        _ffn_kernel,
        grid=(NE,),
        in_specs=[
            pl.BlockSpec((CAP, D), lambda e: (e, 0)),
            pl.BlockSpec((1, D, DFF), lambda e: (e, 0, 0)),
            pl.BlockSpec((1, 1, DFF), lambda e: (e, 0, 0)),
            pl.BlockSpec((1, DFF, D), lambda e: (e, 0, 0)),
            pl.BlockSpec((1, 1, D), lambda e: (e, 0, 0)),
        ],
        out_specs=pl.BlockSpec((CAP, D), lambda e: (e, 0)),
        out_shape=jax.ShapeDtypeStruct((NSLOT, D), _F32),
        compiler_params=pltpu.CompilerParams(dimension_semantics=("parallel",)),
    )(buf, w1, b1, w2, b2)


def _lnout_kernel(gath_ref, scale_ref, a_ref, g_ref, b_ref, o_ref, obf_ref):
    ffn = gath_ref[...] * scale_ref[...]
    out = _ln(ffn + a_ref[...], g_ref[...], b_ref[...])
    o_ref[...] = out
    obf_ref[...] = out.astype(_BF)


def _lnout(gath, scale, apad, g, b):
    return pl.pallas_call(
        _lnout_kernel,
        grid=(T // TB,),
        in_specs=[
            pl.BlockSpec((TB, D), lambda i: (i, 0)),
            pl.BlockSpec((TB, 1), lambda i: (i, 0)),
            pl.BlockSpec((TB, D), lambda i: (i, 0)),
            pl.BlockSpec((1, D), lambda i: (0, 0)),
            pl.BlockSpec((1, D), lambda i: (0, 0)),
        ],
        out_specs=(
            pl.BlockSpec((TB, D), lambda i: (i, 0)),
            pl.BlockSpec((TB, D), lambda i: (i, 0)),
        ),
        out_shape=(
            jax.ShapeDtypeStruct((T, D), _F32),
            jax.ShapeDtypeStruct((T, D), _BF),
        ),
        compiler_params=pltpu.CompilerParams(dimension_semantics=("arbitrary",)),
    )(gath, scale, apad, g, b)


def kernel(hidden_states, W_emb, b_emb, Wq, bq, Wk, bk, Wv, bv, Wo, bo,
           ln_attn_g, ln_attn_b, Wr, W1, b1, W2, b2, ln_out_g, ln_out_b):
    x = hidden_states.reshape(T, EMBD)
    r2 = lambda v: v.reshape(1, -1)
    b1h = b1.reshape(NE, 1, DFF)
    b2h = b2.reshape(NE, 1, D)
    h, hbf = _embed(x, W_emb, r2(b_emb))
    losses = []
    for _ in range(NLAYERS):
        ctx = _attn(hbf, Wq, r2(bq), Wk, r2(bk), Wv, r2(bv))
        a, code, code2, scale, loss = _router(
            ctx, Wo, r2(bo), h, r2(ln_attn_g), r2(ln_attn_b), Wr)
        sc_dispatch, sc_combine = _sc_kernels()
        buf = sc_dispatch(a, code.reshape(T))
        y = _ffn(buf, W1, b1h, W2, b2h)
        gath = sc_combine(y, code2.reshape(T))
        h, hbf = _lnout(gath, scale, a, r2(ln_out_g), r2(ln_out_b))
        losses.append(loss[0, 0])
    return h.reshape(1, T, D), jnp.stack(losses)
